# baseline (speedup 1.0000x reference)
import functools

import jax
import jax.numpy as jnp
from jax import lax
from jax.experimental import pallas as pl
from jax.experimental.pallas import tpu as pltpu
from jax.experimental.pallas import tpu_sc as plsc

_D = 256
_MW = 50


def _build_sc_gather(N, T):
    info = plsc.get_sparse_core_info()
    NC, NS, L = info.num_cores, info.num_subcores, info.num_lanes
    NW = NC * NS
    NPW = N // NW
    RPW = NPW * _MW
    CH = 128
    NCH = RPW // CH
    ZROW = T

    mesh = plsc.VectorSubcoreMesh(core_axis_name="c", subcore_axis_name="s")

    @functools.partial(
        pl.kernel,
        mesh=mesh,
        out_type=jax.ShapeDtypeStruct((N * _MW, _D), jnp.float32),
        scratch_types=[
            pltpu.VMEM((NPW,), jnp.int32),
            pltpu.VMEM((NPW,), jnp.int32),
            pltpu.VMEM((RPW + 2 * L,), jnp.int32),
            pltpu.VMEM((CH, _D), jnp.float32),
            pltpu.SemaphoreType.DMA,
        ],
    )
    def sc_gather(starts_hbm, ends_hbm, table_hbm, out_hbm,
                  starts_v, ends_v, idx_v, buf, gsem):
        wid = lax.axis_index("s") * NC + lax.axis_index("c")
        nbase = pl.multiple_of(wid * NPW, NPW)
        pltpu.sync_copy(starts_hbm.at[pl.ds(nbase, NPW)], starts_v)
        pltpu.sync_copy(ends_hbm.at[pl.ds(nbase, NPW)], ends_v)

        lane = lax.iota(jnp.int32, L)
        njc = (_MW + L - 1) // L

        def group_body(g, _):
            goff = pl.multiple_of(g * L, L)
            sv = starts_v[pl.ds(goff, L)]
            ev = ends_v[pl.ds(goff, L)]
            gbase = g * (L * _MW)
            for i in range(L):
                s = sv[i]
                e = ev[i]
                meta = e < 0
                eff = jnp.where(meta, T - 1, s)
                cnt = jnp.where(meta, 1, e - s + 1)
                for jc in range(njc):
                    jv = jc * L + lane
                    idx = jnp.where(jv < cnt, eff + jv, ZROW)
                    idx_v[pl.ds(gbase + (i * _MW + jc * L), L)] = idx
            return 0

        lax.fori_loop(0, NPW // L, group_body, 0)

        rbase = wid * RPW

        def chunk_body(c, _):
            ioff = pl.multiple_of(c * CH, CH)
            pltpu.async_copy(
                table_hbm.at[idx_v.at[pl.ds(ioff, CH)]], buf, gsem).wait()
            pltpu.sync_copy(
                buf, out_hbm.at[pl.ds(pl.multiple_of(rbase + ioff, CH), CH)])
            return 0

        lax.fori_loop(0, NCH, chunk_body, 0)

    return sc_gather


def kernel(batched_nodes, batched_bi_lstm_outputs):
    nodes0 = batched_nodes[0]
    H = batched_bi_lstm_outputs[0]
    N = nodes0.shape[0]
    T = H.shape[0]
    starts = nodes0[:, 0]
    ends = nodes0[:, 1]
    table = jnp.concatenate([H, jnp.zeros((1, _D), H.dtype)], axis=0)
    out = _build_sc_gather(N, T)(starts, ends, table)
    return out.reshape(1, N, _MW, _D)
